# Initial kernel scaffold; baseline (speedup 1.0000x reference)
import jax
import jax.numpy as jnp
from jax.experimental import pallas as pl


def kernel(motion, destination_frame):
    raise NotImplementedError("write your pallas kernel here")


# speedup vs baseline: 13.1105x; 13.1105x over previous
import jax
import jax.numpy as jnp
from jax import lax
from jax.experimental import pallas as pl
from jax.experimental.pallas import tpu as pltpu
from jax.experimental.pallas import tpu_sc as plsc

H = 512
W = 512
P = H * W
NC = 2
NS = 16
NW = NC * NS
PPW = P // NW
CH = PPW // 16
MAGIC = 8388608.0


def _sc_euler(tabx_hbm, taby_hbm, nv_hbm, out_hbm, nv, idxb, gx, gy,
              dcx, dcy, ax, ay, sem):
    c = lax.axis_index("c")
    s = lax.axis_index("s")
    wid = c * NS + s
    base = wid * PPW

    pltpu.sync_copy(nv_hbm, nv)
    d1 = pltpu.async_copy(tabx_hbm.at[pl.ds(base, PPW)], gx, sem)
    d2 = pltpu.async_copy(taby_hbm.at[pl.ds(base, PPW)], gy, sem)
    d1.wait()
    d2.wait()

    n = nv[...][0]
    iota = lax.iota(jnp.int32, 16)

    def init_chunk(j, _):
        sl = pl.ds(j * 16, 16)
        p = base + j * 16 + iota
        cx = (p & (W - 1)).astype(jnp.float32)
        cy = (p >> 9).astype(jnp.float32)
        dcx[sl] = cx
        dcy[sl] = cy
        ax[sl] = -gx[sl]
        ay[sl] = -gy[sl]
        return 0

    lax.fori_loop(0, CH, init_chunk, 0)

    def compute_pass(j, _):
        sl = pl.ds(j * 16, 16)
        p = base + j * 16 + iota
        cx = (p & (W - 1)).astype(jnp.float32)
        cy = (p >> 9).astype(jnp.float32)
        gxv = gx[sl]
        gyv = gy[sl]
        ax[sl] = ax[sl] + gxv
        ay[sl] = ay[sl] + gyv
        dxl = dcx[sl]
        dyl = dcy[sl]
        mb0 = dxl < -0.5
        tx = jnp.where(mb0, cx, dxl) + gxv
        ty = dyl + gyv
        oob = (tx > W - 1.0) | (tx < 0.0) | (ty > H - 1.0) | (ty < 0.0)
        m = mb0 | oob
        dxe = jnp.where(m, cx, tx)
        dye = jnp.where(m, cy, ty)
        dcx[sl] = jnp.where(m, -1.0 - cx, tx)
        dcy[sl] = dye
        rx = ((dxe + MAGIC) - MAGIC).astype(jnp.int32)
        ry = ((dye + MAGIC) - MAGIC).astype(jnp.int32)
        idxb[0, sl] = (ry << 9) | rx
        return 0

    def iter_body(it, _):
        lax.fori_loop(0, CH, compute_pass, 0)
        g1 = pltpu.async_copy(tabx_hbm.at[idxb.at[0]], gx, sem)
        g2 = pltpu.async_copy(taby_hbm.at[idxb.at[0]], gy, sem)
        g1.wait()
        g2.wait()
        return 0

    lax.fori_loop(0, n, iter_body, 0)

    def fin_chunk(j, _):
        sl = pl.ds(j * 16, 16)
        ax[sl] = ax[sl] + gx[sl]
        ay[sl] = ay[sl] + gy[sl]
        return 0

    lax.fori_loop(0, CH, fin_chunk, 0)

    pltpu.sync_copy(ax, out_hbm.at[0, pl.ds(base, PPW)])
    pltpu.sync_copy(ay, out_hbm.at[1, pl.ds(base, PPW)])


@jax.jit
def kernel(motion, destination_frame):
    tabx = motion[0, 0].reshape(P).astype(jnp.float32)
    taby = motion[0, 1].reshape(P).astype(jnp.float32)
    nvec = jnp.broadcast_to(destination_frame.astype(jnp.int32).reshape(1), (16,))
    mesh = plsc.VectorSubcoreMesh(core_axis_name="c", subcore_axis_name="s")
    out = pl.kernel(
        _sc_euler,
        out_type=jax.ShapeDtypeStruct((2, P), jnp.float32),
        mesh=mesh,
        scratch_types=[
            pltpu.VMEM((16,), jnp.int32),
            pltpu.VMEM((1, PPW), jnp.int32),
            pltpu.VMEM((PPW,), jnp.float32),
            pltpu.VMEM((PPW,), jnp.float32),
            pltpu.VMEM((PPW,), jnp.float32),
            pltpu.VMEM((PPW,), jnp.float32),
            pltpu.VMEM((PPW,), jnp.float32),
            pltpu.VMEM((PPW,), jnp.float32),
            pltpu.SemaphoreType.DMA,
        ],
    )(tabx, taby, nvec)
    return out.reshape(1, 2, H, W)


# speedup vs baseline: 13.8211x; 1.0542x over previous
import jax
import jax.numpy as jnp
from jax import lax
from jax.experimental import pallas as pl
from jax.experimental.pallas import tpu as pltpu
from jax.experimental.pallas import tpu_sc as plsc

H = 512
W = 512
P = H * W
NC = 2
NS = 16
NW = NC * NS
PPW = P // NW
CH = PPW // 16
MAGIC = 8388608.0


def _sc_euler(tabx_hbm, taby_hbm, nv_hbm, out_hbm, nv, idxb, gx, gy,
              dcx, dcy, ax, ay, sem):
    c = lax.axis_index("c")
    s = lax.axis_index("s")
    wid = c * NS + s
    base = wid * PPW

    pltpu.sync_copy(nv_hbm, nv)
    d1 = pltpu.async_copy(tabx_hbm.at[pl.ds(base, PPW)], gx, sem)
    d2 = pltpu.async_copy(taby_hbm.at[pl.ds(base, PPW)], gy, sem)
    d1.wait()
    d2.wait()

    n = nv[...][0]
    iota = lax.iota(jnp.int32, 16)

    @plsc.parallel_loop(0, CH, unroll=4)
    def init_chunk(j):
        sl = pl.ds(j * 16, 16)
        p = base + j * 16 + iota
        cx = (p & (W - 1)).astype(jnp.float32)
        cy = (p >> 9).astype(jnp.float32)
        dcx[sl] = cx
        dcy[sl] = cy
        ax[sl] = -gx[sl]
        ay[sl] = -gy[sl]

    def compute_pass(j):
        sl = pl.ds(j * 16, 16)
        p = base + j * 16 + iota
        cx = (p & (W - 1)).astype(jnp.float32)
        cy = (p >> 9).astype(jnp.float32)
        gxv = gx[sl]
        gyv = gy[sl]
        ax[sl] = ax[sl] + gxv
        ay[sl] = ay[sl] + gyv
        dxl = dcx[sl]
        dyl = dcy[sl]
        mb0 = dxl < -0.5
        tx = jnp.where(mb0, cx, dxl) + gxv
        ty = dyl + gyv
        oob = (tx > W - 1.0) | (tx < 0.0) | (ty > H - 1.0) | (ty < 0.0)
        m = mb0 | oob
        dxe = jnp.where(m, cx, tx)
        dye = jnp.where(m, cy, ty)
        dcx[sl] = jnp.where(m, -1.0 - cx, tx)
        dcy[sl] = dye
        rx = ((dxe + MAGIC) - MAGIC).astype(jnp.int32)
        ry = ((dye + MAGIC) - MAGIC).astype(jnp.int32)
        idxb[0, sl] = (ry << 9) | rx

    def iter_body(it, _):
        plsc.parallel_loop(0, CH, unroll=4)(compute_pass)
        g1 = pltpu.async_copy(tabx_hbm.at[idxb.at[0]], gx, sem)
        g2 = pltpu.async_copy(taby_hbm.at[idxb.at[0]], gy, sem)
        g1.wait()
        g2.wait()
        return 0

    lax.fori_loop(0, n, iter_body, 0)

    @plsc.parallel_loop(0, CH, unroll=4)
    def fin_chunk(j):
        sl = pl.ds(j * 16, 16)
        ax[sl] = ax[sl] + gx[sl]
        ay[sl] = ay[sl] + gy[sl]

    pltpu.sync_copy(ax, out_hbm.at[0, pl.ds(base, PPW)])
    pltpu.sync_copy(ay, out_hbm.at[1, pl.ds(base, PPW)])


@jax.jit
def kernel(motion, destination_frame):
    tabx = motion[0, 0].reshape(P).astype(jnp.float32)
    taby = motion[0, 1].reshape(P).astype(jnp.float32)
    nvec = jnp.broadcast_to(destination_frame.astype(jnp.int32).reshape(1), (16,))
    mesh = plsc.VectorSubcoreMesh(core_axis_name="c", subcore_axis_name="s")
    out = pl.kernel(
        _sc_euler,
        out_type=jax.ShapeDtypeStruct((2, P), jnp.float32),
        mesh=mesh,
        scratch_types=[
            pltpu.VMEM((16,), jnp.int32),
            pltpu.VMEM((1, PPW), jnp.int32),
            pltpu.VMEM((PPW,), jnp.float32),
            pltpu.VMEM((PPW,), jnp.float32),
            pltpu.VMEM((PPW,), jnp.float32),
            pltpu.VMEM((PPW,), jnp.float32),
            pltpu.VMEM((PPW,), jnp.float32),
            pltpu.VMEM((PPW,), jnp.float32),
            pltpu.SemaphoreType.DMA,
        ],
    )(tabx, taby, nvec)
    return out.reshape(1, 2, H, W)


# speedup vs baseline: 15.6406x; 1.1317x over previous
import jax
import jax.numpy as jnp
from jax import lax
from jax.experimental import pallas as pl
from jax.experimental.pallas import tpu as pltpu
from jax.experimental.pallas import tpu_sc as plsc

H = 512
W = 512
P = H * W
NC = 2
NS = 16
NW = NC * NS
PPW = P // NW
HALF = PPW // 2
CHH = HALF // 16
MAGIC = 8388608.0


def _sc_euler(tabx_hbm, taby_hbm, nv_hbm, out_hbm, nv, idxa, idxb, gx, gy,
              dcx, dcy, ax, ay, sema, semb):
    c = lax.axis_index("c")
    s = lax.axis_index("s")
    wid = c * NS + s
    base = wid * PPW

    pltpu.sync_copy(nv_hbm, nv)
    n = nv[...][0]
    iota = lax.iota(jnp.int32, 16)

    pltpu.async_copy(tabx_hbm.at[pl.ds(base, HALF)], gx.at[pl.ds(0, HALF)], sema)
    pltpu.async_copy(taby_hbm.at[pl.ds(base, HALF)], gy.at[pl.ds(0, HALF)], sema)
    pltpu.async_copy(tabx_hbm.at[pl.ds(base + HALF, HALF)], gx.at[pl.ds(HALF, HALF)], semb)
    pltpu.async_copy(taby_hbm.at[pl.ds(base + HALF, HALF)], gy.at[pl.ds(HALF, HALF)], semb)

    def make_pass(off, idxr, first):
        def compute_chunk(j):
            sl = pl.ds(off + j * 16, 16)
            p = base + off + j * 16 + iota
            cx = (p & (W - 1)).astype(jnp.float32)
            cy = (p >> 9).astype(jnp.float32)
            gxv = gx[sl]
            gyv = gy[sl]
            if first:
                ax[sl] = jnp.zeros((16,), jnp.float32)
                ay[sl] = jnp.zeros((16,), jnp.float32)
                tx = cx + gxv
                ty = cy + gyv
                mb0 = None
            else:
                ax[sl] = ax[sl] + gxv
                ay[sl] = ay[sl] + gyv
                dxl = dcx[sl]
                dyl = dcy[sl]
                mb0 = dxl < -0.5
                tx = jnp.where(mb0, cx, dxl) + gxv
                ty = dyl + gyv
            oob = (tx > W - 1.0) | (tx < 0.0) | (ty > H - 1.0) | (ty < 0.0)
            m = oob if first else (mb0 | oob)
            dxe = jnp.where(m, cx, tx)
            dye = jnp.where(m, cy, ty)
            dcx[sl] = jnp.where(m, -1.0 - cx, tx)
            dcy[sl] = dye
            rx = ((dxe + MAGIC) - MAGIC).astype(jnp.int32)
            ry = ((dye + MAGIC) - MAGIC).astype(jnp.int32)
            idxr[0, pl.ds(j * 16, 16)] = (ry << 9) | rx
        return compute_chunk

    pass_a0 = make_pass(0, idxa, True)
    pass_b0 = make_pass(HALF, idxb, True)
    pass_a = make_pass(0, idxa, False)
    pass_b = make_pass(HALF, idxb, False)

    def wait_half(sem, off):
        pltpu.make_async_copy(tabx_hbm.at[pl.ds(0, HALF)], gx.at[pl.ds(off, HALF)], sem).wait()
        pltpu.make_async_copy(taby_hbm.at[pl.ds(0, HALF)], gy.at[pl.ds(off, HALF)], sem).wait()

    def fire_half(sem, off, idxr):
        pltpu.async_copy(tabx_hbm.at[idxr.at[0]], gx.at[pl.ds(off, HALF)], sem)
        pltpu.async_copy(taby_hbm.at[idxr.at[0]], gy.at[pl.ds(off, HALF)], sem)

    wait_half(sema, 0)
    plsc.parallel_loop(0, CHH, unroll=4)(pass_a0)
    fire_half(sema, 0, idxa)
    wait_half(semb, HALF)
    plsc.parallel_loop(0, CHH, unroll=4)(pass_b0)
    fire_half(semb, HALF, idxb)

    def iter_body(it, _):
        wait_half(sema, 0)
        plsc.parallel_loop(0, CHH, unroll=4)(pass_a)
        fire_half(sema, 0, idxa)
        wait_half(semb, HALF)
        plsc.parallel_loop(0, CHH, unroll=4)(pass_b)
        fire_half(semb, HALF, idxb)
        return 0

    lax.fori_loop(1, n, iter_body, 0)

    wait_half(sema, 0)
    wait_half(semb, HALF)

    @plsc.parallel_loop(0, PPW // 16, unroll=4)
    def fin_chunk(j):
        sl = pl.ds(j * 16, 16)
        ax[sl] = ax[sl] + gx[sl]
        ay[sl] = ay[sl] + gy[sl]

    pltpu.sync_copy(ax, out_hbm.at[0, pl.ds(base, PPW)])
    pltpu.sync_copy(ay, out_hbm.at[1, pl.ds(base, PPW)])


@jax.jit
def kernel(motion, destination_frame):
    tabx = motion[0, 0].reshape(P).astype(jnp.float32)
    taby = motion[0, 1].reshape(P).astype(jnp.float32)
    nvec = jnp.broadcast_to(destination_frame.astype(jnp.int32).reshape(1), (16,))
    mesh = plsc.VectorSubcoreMesh(core_axis_name="c", subcore_axis_name="s")
    out = pl.kernel(
        _sc_euler,
        out_type=jax.ShapeDtypeStruct((2, P), jnp.float32),
        mesh=mesh,
        scratch_types=[
            pltpu.VMEM((16,), jnp.int32),
            pltpu.VMEM((1, HALF), jnp.int32),
            pltpu.VMEM((1, HALF), jnp.int32),
            pltpu.VMEM((PPW,), jnp.float32),
            pltpu.VMEM((PPW,), jnp.float32),
            pltpu.VMEM((PPW,), jnp.float32),
            pltpu.VMEM((PPW,), jnp.float32),
            pltpu.VMEM((PPW,), jnp.float32),
            pltpu.VMEM((PPW,), jnp.float32),
            pltpu.SemaphoreType.DMA,
            pltpu.SemaphoreType.DMA,
        ],
    )(tabx, taby, nvec)
    return out.reshape(1, 2, H, W)


# speedup vs baseline: 45.8309x; 2.9302x over previous
import jax
import jax.numpy as jnp
from jax import lax
from jax.experimental import pallas as pl
from jax.experimental.pallas import tpu as pltpu
from jax.experimental.pallas import tpu_sc as plsc

H = 512
W = 512
P = H * W
NC = 2
NS = 16
NW = NC * NS
PPW = P // NW
NB = 2
BATCH = PPW // NB
BHALF = BATCH // 2
CHH = BHALF // 16
MAGIC = 8388608.0


def _sc_euler(tabx_hbm, taby_hbm, nv_hbm, out_hbm, nv, idxa, idxb, gx, gy,
              dcx, dcy, ax, ay, tabsx, tabsy, sema, semb):
    c = lax.axis_index("c")
    s = lax.axis_index("s")
    wid = c * NS + s
    base = wid * PPW

    seg = P // NS
    for tab_hbm, tabs in ((tabx_hbm, tabsx), (taby_hbm, tabsy)):
        for h in range(seg // BATCH):
            off = s * seg + h * BATCH
            pltpu.sync_copy(tab_hbm.at[pl.ds(off, BATCH)], gx)
            pltpu.sync_copy(gx, tabs.at[pl.ds(off, BATCH)])
    plsc.subcore_barrier()

    pltpu.sync_copy(nv_hbm, nv)
    n = nv[...][0]
    iota = lax.iota(jnp.int32, 16)

    def wait_half(sem, off):
        pltpu.make_async_copy(tabx_hbm.at[pl.ds(0, BHALF)], gx.at[pl.ds(off, BHALF)], sem).wait()
        pltpu.make_async_copy(taby_hbm.at[pl.ds(0, BHALF)], gy.at[pl.ds(off, BHALF)], sem).wait()

    def fire_half(sem, off, idxr):
        pltpu.async_copy(tabsx.at[idxr.at[0]], gx.at[pl.ds(off, BHALF)], sem)
        pltpu.async_copy(tabsy.at[idxr.at[0]], gy.at[pl.ds(off, BHALF)], sem)

    for b in range(NB):
        bbase = base + b * BATCH

        def make_pass(off, idxr, first):
            def compute_chunk(j):
                sl = pl.ds(off + j * 16, 16)
                p = bbase + off + j * 16 + iota
                cx = (p & (W - 1)).astype(jnp.float32)
                cy = (p >> 9).astype(jnp.float32)
                gxv = gx[sl]
                gyv = gy[sl]
                if first:
                    ax[sl] = jnp.zeros((16,), jnp.float32)
                    ay[sl] = jnp.zeros((16,), jnp.float32)
                    tx = cx + gxv
                    ty = cy + gyv
                    mb0 = None
                else:
                    ax[sl] = ax[sl] + gxv
                    ay[sl] = ay[sl] + gyv
                    dxl = dcx[sl]
                    dyl = dcy[sl]
                    mb0 = dxl < -0.5
                    tx = jnp.where(mb0, cx, dxl) + gxv
                    ty = dyl + gyv
                oob = (tx > W - 1.0) | (tx < 0.0) | (ty > H - 1.0) | (ty < 0.0)
                m = oob if first else (mb0 | oob)
                dxe = jnp.where(m, cx, tx)
                dye = jnp.where(m, cy, ty)
                dcx[sl] = jnp.where(m, -1.0 - cx, tx)
                dcy[sl] = dye
                rx = ((dxe + MAGIC) - MAGIC).astype(jnp.int32)
                ry = ((dye + MAGIC) - MAGIC).astype(jnp.int32)
                idxr[0, pl.ds(j * 16, 16)] = (ry << 9) | rx
            return compute_chunk

        pass_a0 = make_pass(0, idxa, True)
        pass_b0 = make_pass(BHALF, idxb, True)
        pass_a = make_pass(0, idxa, False)
        pass_b = make_pass(BHALF, idxb, False)

        pltpu.async_copy(tabsx.at[pl.ds(bbase, BHALF)], gx.at[pl.ds(0, BHALF)], sema)
        pltpu.async_copy(tabsy.at[pl.ds(bbase, BHALF)], gy.at[pl.ds(0, BHALF)], sema)
        pltpu.async_copy(tabsx.at[pl.ds(bbase + BHALF, BHALF)], gx.at[pl.ds(BHALF, BHALF)], semb)
        pltpu.async_copy(tabsy.at[pl.ds(bbase + BHALF, BHALF)], gy.at[pl.ds(BHALF, BHALF)], semb)

        wait_half(sema, 0)
        plsc.parallel_loop(0, CHH, unroll=4)(pass_a0)
        fire_half(sema, 0, idxa)
        wait_half(semb, BHALF)
        plsc.parallel_loop(0, CHH, unroll=4)(pass_b0)
        fire_half(semb, BHALF, idxb)

        def iter_body(it, _):
            wait_half(sema, 0)
            plsc.parallel_loop(0, CHH, unroll=4)(pass_a)
            fire_half(sema, 0, idxa)
            wait_half(semb, BHALF)
            plsc.parallel_loop(0, CHH, unroll=4)(pass_b)
            fire_half(semb, BHALF, idxb)
            return 0

        lax.fori_loop(1, n, iter_body, 0)

        wait_half(sema, 0)
        wait_half(semb, BHALF)

        @plsc.parallel_loop(0, BATCH // 16, unroll=4)
        def fin_chunk(j):
            sl = pl.ds(j * 16, 16)
            ax[sl] = ax[sl] + gx[sl]
            ay[sl] = ay[sl] + gy[sl]

        pltpu.sync_copy(ax, out_hbm.at[0, pl.ds(bbase, BATCH)])
        pltpu.sync_copy(ay, out_hbm.at[1, pl.ds(bbase, BATCH)])


@jax.jit
def kernel(motion, destination_frame):
    tabx = motion[0, 0].reshape(P).astype(jnp.float32)
    taby = motion[0, 1].reshape(P).astype(jnp.float32)
    nvec = jnp.broadcast_to(destination_frame.astype(jnp.int32).reshape(1), (16,))
    mesh = plsc.VectorSubcoreMesh(core_axis_name="c", subcore_axis_name="s")
    out = pl.kernel(
        _sc_euler,
        out_type=jax.ShapeDtypeStruct((2, P), jnp.float32),
        mesh=mesh,
        scratch_types=[
            pltpu.VMEM((16,), jnp.int32),
            pltpu.VMEM((1, BHALF), jnp.int32),
            pltpu.VMEM((1, BHALF), jnp.int32),
            pltpu.VMEM((BATCH,), jnp.float32),
            pltpu.VMEM((BATCH,), jnp.float32),
            pltpu.VMEM((BATCH,), jnp.float32),
            pltpu.VMEM((BATCH,), jnp.float32),
            pltpu.VMEM((BATCH,), jnp.float32),
            pltpu.VMEM((BATCH,), jnp.float32),
            pltpu.VMEM_SHARED((P,), jnp.float32),
            pltpu.VMEM_SHARED((P,), jnp.float32),
            pltpu.SemaphoreType.DMA,
            pltpu.SemaphoreType.DMA,
        ],
    )(tabx, taby, nvec)
    return out.reshape(1, 2, H, W)


# speedup vs baseline: 48.1164x; 1.0499x over previous
import jax
import jax.numpy as jnp
from jax import lax
from jax.experimental import pallas as pl
from jax.experimental.pallas import tpu as pltpu
from jax.experimental.pallas import tpu_sc as plsc

H = 512
W = 512
P = H * W
NC = 2
NS = 16
NW = NC * NS
PPW = P // NW
NB = 2
BATCH = PPW // NB
BHALF = BATCH // 2
CHH = BHALF // 16
MAGIC = 8388608.0


def _sc_euler(tabx_hbm, taby_hbm, nv_hbm, out_hbm, nv, idxa, idxb, gx, gy,
              dcx, dcy, ax, ay, tabsx, tabsy, sema, semb):
    c = lax.axis_index("c")
    s = lax.axis_index("s")
    wid = c * NS + s
    base = wid * PPW

    seg = P // NS
    pltpu.async_copy(tabx_hbm.at[pl.ds(s * seg, seg)], tabsx.at[pl.ds(s * seg, seg)], sema).wait()
    pltpu.async_copy(taby_hbm.at[pl.ds(s * seg, seg)], tabsy.at[pl.ds(s * seg, seg)], semb).wait()
    plsc.subcore_barrier()

    pltpu.sync_copy(nv_hbm, nv)
    n = nv[...][0]
    iota = lax.iota(jnp.int32, 16)

    def wait_half(sem, off):
        pltpu.make_async_copy(tabx_hbm.at[pl.ds(0, BHALF)], gx.at[pl.ds(off, BHALF)], sem).wait()
        pltpu.make_async_copy(taby_hbm.at[pl.ds(0, BHALF)], gy.at[pl.ds(off, BHALF)], sem).wait()

    def fire_half(sem, off, idxr):
        pltpu.async_copy(tabsx.at[idxr.at[0]], gx.at[pl.ds(off, BHALF)], sem)
        pltpu.async_copy(tabsy.at[idxr.at[0]], gy.at[pl.ds(off, BHALF)], sem)

    for b in range(NB):
        bbase = base + b * BATCH

        def make_pass(off, idxr, first):
            def compute_chunk(j):
                sl = pl.ds(off + j * 16, 16)
                p = bbase + off + j * 16 + iota
                cx = (p & (W - 1)).astype(jnp.float32)
                cy = (p >> 9).astype(jnp.float32)
                gxv = gx[sl]
                gyv = gy[sl]
                if first:
                    ax[sl] = jnp.zeros((16,), jnp.float32)
                    ay[sl] = jnp.zeros((16,), jnp.float32)
                    tx = cx + gxv
                    ty = cy + gyv
                    mb0 = None
                else:
                    ax[sl] = ax[sl] + gxv
                    ay[sl] = ay[sl] + gyv
                    dxl = dcx[sl]
                    dyl = dcy[sl]
                    mb0 = dxl < -0.5
                    tx = jnp.where(mb0, cx, dxl) + gxv
                    ty = dyl + gyv
                oob = (tx > W - 1.0) | (tx < 0.0) | (ty > H - 1.0) | (ty < 0.0)
                m = oob if first else (mb0 | oob)
                dxe = jnp.where(m, cx, tx)
                dye = jnp.where(m, cy, ty)
                dcx[sl] = jnp.where(m, -1.0 - cx, tx)
                dcy[sl] = dye
                rx = ((dxe + MAGIC) - MAGIC).astype(jnp.int32)
                ry = ((dye + MAGIC) - MAGIC).astype(jnp.int32)
                idxr[0, pl.ds(j * 16, 16)] = (ry << 9) | rx
            return compute_chunk

        pass_a0 = make_pass(0, idxa, True)
        pass_b0 = make_pass(BHALF, idxb, True)
        pass_a = make_pass(0, idxa, False)
        pass_b = make_pass(BHALF, idxb, False)

        pltpu.async_copy(tabsx.at[pl.ds(bbase, BHALF)], gx.at[pl.ds(0, BHALF)], sema)
        pltpu.async_copy(tabsy.at[pl.ds(bbase, BHALF)], gy.at[pl.ds(0, BHALF)], sema)
        pltpu.async_copy(tabsx.at[pl.ds(bbase + BHALF, BHALF)], gx.at[pl.ds(BHALF, BHALF)], semb)
        pltpu.async_copy(tabsy.at[pl.ds(bbase + BHALF, BHALF)], gy.at[pl.ds(BHALF, BHALF)], semb)

        wait_half(sema, 0)
        plsc.parallel_loop(0, CHH, unroll=4)(pass_a0)
        fire_half(sema, 0, idxa)
        wait_half(semb, BHALF)
        plsc.parallel_loop(0, CHH, unroll=4)(pass_b0)
        fire_half(semb, BHALF, idxb)

        def iter_body(it, _):
            wait_half(sema, 0)
            plsc.parallel_loop(0, CHH, unroll=4)(pass_a)
            fire_half(sema, 0, idxa)
            wait_half(semb, BHALF)
            plsc.parallel_loop(0, CHH, unroll=4)(pass_b)
            fire_half(semb, BHALF, idxb)
            return 0

        lax.fori_loop(1, n, iter_body, 0)

        wait_half(sema, 0)
        wait_half(semb, BHALF)

        @plsc.parallel_loop(0, BATCH // 16, unroll=4)
        def fin_chunk(j):
            sl = pl.ds(j * 16, 16)
            ax[sl] = ax[sl] + gx[sl]
            ay[sl] = ay[sl] + gy[sl]

        pltpu.sync_copy(ax, out_hbm.at[0, pl.ds(bbase, BATCH)])
        pltpu.sync_copy(ay, out_hbm.at[1, pl.ds(bbase, BATCH)])


@jax.jit
def kernel(motion, destination_frame):
    tabx = motion[0, 0].reshape(P).astype(jnp.float32)
    taby = motion[0, 1].reshape(P).astype(jnp.float32)
    nvec = jnp.broadcast_to(destination_frame.astype(jnp.int32).reshape(1), (16,))
    mesh = plsc.VectorSubcoreMesh(core_axis_name="c", subcore_axis_name="s")
    out = pl.kernel(
        _sc_euler,
        out_type=jax.ShapeDtypeStruct((2, P), jnp.float32),
        mesh=mesh,
        scratch_types=[
            pltpu.VMEM((16,), jnp.int32),
            pltpu.VMEM((1, BHALF), jnp.int32),
            pltpu.VMEM((1, BHALF), jnp.int32),
            pltpu.VMEM((BATCH,), jnp.float32),
            pltpu.VMEM((BATCH,), jnp.float32),
            pltpu.VMEM((BATCH,), jnp.float32),
            pltpu.VMEM((BATCH,), jnp.float32),
            pltpu.VMEM((BATCH,), jnp.float32),
            pltpu.VMEM((BATCH,), jnp.float32),
            pltpu.VMEM_SHARED((P,), jnp.float32),
            pltpu.VMEM_SHARED((P,), jnp.float32),
            pltpu.SemaphoreType.DMA,
            pltpu.SemaphoreType.DMA,
        ],
    )(tabx, taby, nvec)
    return out.reshape(1, 2, H, W)
